# baseline (speedup 1.0000x reference)
import functools

import jax
import jax.numpy as jnp
from jax import lax
from jax.experimental import pallas as pl
from jax.experimental.pallas import tpu as pltpu, tpu_sc as plsc

N = 10000
NP = 10240
D = 128
H = 128
E = 320000
NC = 2
NS = 16
NW = NC * NS
F = D // NW
CE = 6400
ECNT = E // NW
BLK = 1280
GRID = NP // BLK


def _bn_scale_shift(g, b, m, v):
    s = g / jnp.sqrt(v + 1e-3)
    return s, b - m * s


def _gelu(z):
    return 0.5 * z * (1.0 + lax.erf(z * 0.7071067811865476))


def _prepare_body(xt_ref, w1t_ref, w2t_ref, s1_ref, t1_ref, s2_ref, t2_ref,
                  c1_ref, c2_ref, out_ref):
    xb = xt_ref[...] * s1_ref[...] + t1_ref[...]
    h1 = _gelu(jnp.dot(w1t_ref[...], xb, preferred_element_type=jnp.float32)
               + c1_ref[...])
    hb = h1 * s2_ref[...] + t2_ref[...]
    h = _gelu(jnp.dot(w2t_ref[...], hb, preferred_element_type=jnp.float32)
              + c2_ref[...])
    au = lax.bitcast_convert_type(h[:H // 2].astype(jnp.bfloat16),
                                  jnp.uint16).astype(jnp.int32)
    bu = lax.bitcast_convert_type(h[H // 2:].astype(jnp.bfloat16),
                                  jnp.uint16).astype(jnp.int32)
    out_ref[...] = au | (bu << 16)


def _prepare_ffn_t(xt, w1t, w2t, s1, t1, s2, t2, c1, c2):
    col = pl.BlockSpec((D, 1), lambda i: (0, 0))
    full = pl.BlockSpec((D, D), lambda i: (0, 0))
    return pl.pallas_call(
        _prepare_body,
        grid=(GRID,),
        in_specs=[pl.BlockSpec((D, BLK), lambda i: (0, i)),
                  full, full, col, col, col, col, col, col],
        out_specs=pl.BlockSpec((H // 2, BLK), lambda i: (0, i)),
        out_shape=jax.ShapeDtypeStruct((H // 2, NP), jnp.int32),
    )(xt, w1t, w2t, s1, t1, s2, t2, c1, c2)


NCH = E // CE
CW = 2 * CE


def _edge_body(ht_hbm, epk_hbm, dst_hbm, sums_hbm, cnt_hbm,
               htp0, htp1, acc0, acc1, acc2, acc3, ebuf0, ebuf1,
               cnt_buf, cdst_buf, sem0, sem1, semc):
    wid = lax.axis_index("s") * NC + lax.axis_index("c")
    p0 = 2 * wid

    coff = pl.multiple_of(wid * ECNT, 8)
    pltpu.async_copy(dst_hbm.at[pl.ds(coff, ECNT)], cdst_buf, semc)

    pltpu.sync_copy(ht_hbm.at[pl.ds(pl.multiple_of(p0 * NP, 8), NP)], htp0)
    pltpu.sync_copy(ht_hbm.at[pl.ds(pl.multiple_of((p0 + 1) * NP, 8), NP)],
                    htp1)

    for a in (acc0, acc1, acc2, acc3):
        @plsc.parallel_loop(0, NP, step=16, unroll=8)
        def _zero_acc(j, a=a):
            a[pl.ds(j, 16)] = jnp.zeros((16,), jnp.float32)

    pltpu.async_copy(epk_hbm.at[pl.ds(0, CW)], ebuf0, sem0)
    pltpu.async_copy(epk_hbm.at[pl.ds(CW, CW)], ebuf1, sem1)

    hi_mask = jnp.full((16,), -65536, jnp.int32)
    lo_mask = jnp.full((16,), 65535, jnp.int32)

    def _pair(g, _):
        for b, (ebuf, sem) in enumerate(((ebuf0, sem0), (ebuf1, sem1))):
            k = 2 * g + b
            pltpu.make_async_copy(epk_hbm.at[pl.ds(0, CW)], ebuf, sem).wait()

            @plsc.parallel_loop(0, CE, step=16, unroll=16)
            def _win(i):
                sd = ebuf[pl.ds(i, 16)]
                wt = plsc.bitcast(ebuf[pl.ds(CE + i, 16)], jnp.float32)
                s = sd & lo_mask
                d = lax.shift_right_logical(sd, 16)
                for htp, accl, acch in ((htp0, acc0, acc1),
                                        (htp1, acc2, acc3)):
                    g2 = plsc.load_gather(htp, [s])
                    lo = plsc.bitcast(lax.shift_left(g2, 16), jnp.float32)
                    hi = plsc.bitcast(g2 & hi_mask, jnp.float32)
                    plsc.addupdate_scatter(accl, [d], lo * wt)
                    plsc.addupdate_scatter(acch, [d], hi * wt)

            off = pl.multiple_of((k + 2) * CW, 8)
            pltpu.async_copy(epk_hbm.at[pl.ds(off, CW)], ebuf, sem)
        return 0
    lax.fori_loop(0, NCH // 2, _pair, 0)
    pltpu.make_async_copy(epk_hbm.at[pl.ds(0, CW)], ebuf0, sem0).wait()
    pltpu.make_async_copy(epk_hbm.at[pl.ds(0, CW)], ebuf1, sem1).wait()

    for a, frow in ((acc0, p0), (acc1, p0 + H // 2),
                    (acc2, p0 + 1), (acc3, p0 + 1 + H // 2)):
        pltpu.sync_copy(a, sums_hbm.at[pl.ds(pl.multiple_of(frow * NP, 8),
                                             NP)])

    pltpu.make_async_copy(dst_hbm.at[pl.ds(0, ECNT)], cdst_buf, semc).wait()

    @plsc.parallel_loop(0, NP, step=16, unroll=8)
    def _zero_cnt(j):
        cnt_buf[pl.ds(j, 16)] = jnp.zeros((16,), jnp.float32)

    ones = jnp.full((16,), 1.0, jnp.float32)

    @plsc.parallel_loop(0, ECNT, step=16, unroll=8)
    def _cwin(i):
        d = cdst_buf[pl.ds(i, 16)]
        plsc.addupdate_scatter(cnt_buf, [d], ones)

    pltpu.sync_copy(cnt_buf, cnt_hbm.at[pl.ds(pl.multiple_of(wid * NP, 8), NP)])


@functools.cache
def _edge_kernel():
    return pl.kernel(
        _edge_body,
        out_type=[jax.ShapeDtypeStruct((D * NP,), jnp.float32),
                  jax.ShapeDtypeStruct((NW * NP,), jnp.float32)],
        mesh=plsc.VectorSubcoreMesh(core_axis_name="c", subcore_axis_name="s",
                                    num_cores=NC, num_subcores=NS),
        compiler_params=pltpu.CompilerParams(needs_layout_passes=False),
        scratch_types=[pltpu.VMEM((NP,), jnp.int32),
                       pltpu.VMEM((NP,), jnp.int32),
                       pltpu.VMEM((NP,), jnp.float32),
                       pltpu.VMEM((NP,), jnp.float32),
                       pltpu.VMEM((NP,), jnp.float32),
                       pltpu.VMEM((NP,), jnp.float32),
                       pltpu.VMEM((CW,), jnp.int32),
                       pltpu.VMEM((CW,), jnp.int32),
                       pltpu.VMEM((NP,), jnp.float32),
                       pltpu.VMEM((ECNT,), jnp.int32),
                       pltpu.SemaphoreType.DMA,
                       pltpu.SemaphoreType.DMA,
                       pltpu.SemaphoreType.DMA])


def _update_body(xt_ref, sums_ref, cntp_ref, uw1xt_ref, uw1at_ref, uw2t_ref,
                 s1x_ref, t1x_ref, s1a_ref, t1a_ref, s2_ref, t2_ref,
                 uc1_ref, uc2_ref, out_ref):
    cnt = jnp.sum(cntp_ref[...], axis=0, keepdims=True)
    agg = sums_ref[...] / jnp.maximum(cnt, 1.0)
    xb = xt_ref[...] * s1x_ref[...] + t1x_ref[...]
    ab = agg * s1a_ref[...] + t1a_ref[...]
    z1 = (jnp.dot(uw1xt_ref[...], xb, preferred_element_type=jnp.float32)
          + jnp.dot(uw1at_ref[...], ab, preferred_element_type=jnp.float32)
          + uc1_ref[...])
    h1 = _gelu(z1)
    hb = h1 * s2_ref[...] + t2_ref[...]
    out_ref[...] = _gelu(jnp.dot(uw2t_ref[...], hb,
                                 preferred_element_type=jnp.float32)
                         + uc2_ref[...])


def _update_ffn_t(xt, sums_t, cntp, uw1xt, uw1at, uw2t,
                  s1x, t1x, s1a, t1a, s2, t2, uc1, uc2):
    col = pl.BlockSpec((H, 1), lambda i: (0, 0))
    full = pl.BlockSpec((H, H), lambda i: (0, 0))
    blk = pl.BlockSpec((D, BLK), lambda i: (0, i))
    return pl.pallas_call(
        _update_body,
        grid=(GRID,),
        in_specs=[blk, blk, pl.BlockSpec((NW, BLK), lambda i: (0, i)),
                  full, full, full,
                  col, col, col, col, col, col, col, col],
        out_specs=pl.BlockSpec((H, BLK), lambda i: (0, i)),
        out_shape=jax.ShapeDtypeStruct((H, NP), jnp.float32),
    )(xt, sums_t, cntp, uw1xt, uw1at, uw2t,
      s1x, t1x, s1a, t1a, s2, t2, uc1, uc2)


def kernel(x, edges, edge_weights, g1, b1, m1, v1, W1, c1, g2, b2, m2, v2,
           W2, c2, ug1, ub1, um1, uv1, UW1, uc1, ug2, ub2, um2, uv2, UW2, uc2):
    xt = jnp.pad(x.T, ((0, 0), (0, NP - N)))

    def colv(p):
        return p.reshape(-1, 1)

    s1, t1 = _bn_scale_shift(g1, b1, m1, v1)
    s2, t2 = _bn_scale_shift(g2, b2, m2, v2)
    ht = _prepare_ffn_t(xt, W1.T, W2.T, colv(s1), colv(t1), colv(s2),
                        colv(t2), colv(c1), colv(c2))

    wbits = lax.bitcast_convert_type(edge_weights, jnp.int32)
    sd = edges[1] | (edges[0] << 16)
    epk = jnp.stack([sd.reshape(NCH, CE), wbits.reshape(NCH, CE)],
                    axis=1).reshape(-1)
    epk = jnp.pad(epk, (0, 2 * CW))
    sums_flat, cnt_flat = _edge_kernel()(ht.reshape(-1), epk, edges[0])

    us1, ut1 = _bn_scale_shift(ug1, ub1, um1, uv1)
    us2, ut2 = _bn_scale_shift(ug2, ub2, um2, uv2)
    out_t = _update_ffn_t(
        xt, sums_flat.reshape(D, NP), cnt_flat.reshape(NW, NP),
        UW1[:D].T, UW1[D:].T, UW2.T,
        colv(us1[:D]), colv(ut1[:D]), colv(us1[D:]), colv(ut1[D:]),
        colv(us2), colv(ut2), colv(uc1), colv(uc2))
    return out_t[:, :N].T


# speedup vs baseline: 1.0094x; 1.0094x over previous
import functools

import jax
import jax.numpy as jnp
from jax import lax
from jax.experimental import pallas as pl
from jax.experimental.pallas import tpu as pltpu, tpu_sc as plsc

N = 10000
NP = 10240
D = 128
H = 128
E = 320000
NC = 2
NS = 16
NW = NC * NS
F = D // NW
CE = 6400
ECNT = E // NW
BLK = 1280
GRID = NP // BLK


def _bn_scale_shift(g, b, m, v):
    s = g / jnp.sqrt(v + 1e-3)
    return s, b - m * s


def _gelu(z):
    return 0.5 * z * (1.0 + lax.erf(z * 0.7071067811865476))


def _prepare_body(xt_ref, w1t_ref, w2t_ref, s1_ref, t1_ref, s2_ref, t2_ref,
                  c1_ref, c2_ref, out_ref):
    xb = xt_ref[...] * s1_ref[...] + t1_ref[...]
    h1 = _gelu(jnp.dot(w1t_ref[...], xb, preferred_element_type=jnp.float32)
               + c1_ref[...])
    hb = h1 * s2_ref[...] + t2_ref[...]
    h = _gelu(jnp.dot(w2t_ref[...], hb, preferred_element_type=jnp.float32)
              + c2_ref[...])
    au = lax.bitcast_convert_type(h[:H // 2].astype(jnp.bfloat16),
                                  jnp.uint16).astype(jnp.int32)
    bu = lax.bitcast_convert_type(h[H // 2:].astype(jnp.bfloat16),
                                  jnp.uint16).astype(jnp.int32)
    out_ref[...] = au | (bu << 16)


def _prepare_ffn_t(xt, w1t, w2t, s1, t1, s2, t2, c1, c2):
    col = pl.BlockSpec((D, 1), lambda i: (0, 0))
    full = pl.BlockSpec((D, D), lambda i: (0, 0))
    return pl.pallas_call(
        _prepare_body,
        grid=(GRID,),
        in_specs=[pl.BlockSpec((D, BLK), lambda i: (0, i)),
                  full, full, col, col, col, col, col, col],
        out_specs=pl.BlockSpec((H // 2, BLK), lambda i: (0, i)),
        out_shape=jax.ShapeDtypeStruct((H // 2, NP), jnp.int32),
    )(xt, w1t, w2t, s1, t1, s2, t2, c1, c2)


NCH = E // CE
CW = 2 * CE


def _edge_body(ht_hbm, epk_hbm, dst_hbm, sums_hbm, cnt_hbm,
               htp0, htp1, acc0, acc1, acc2, acc3, ebuf0, ebuf1,
               cnt_buf, cdst_buf, sem0, sem1, semc):
    wid = lax.axis_index("s") * NC + lax.axis_index("c")
    p0 = 2 * wid

    coff = pl.multiple_of(wid * ECNT, 8)
    pltpu.async_copy(dst_hbm.at[pl.ds(coff, ECNT)], cdst_buf, semc)

    pltpu.sync_copy(ht_hbm.at[pl.ds(pl.multiple_of(p0 * NP, 8), NP)], htp0)
    pltpu.sync_copy(ht_hbm.at[pl.ds(pl.multiple_of((p0 + 1) * NP, 8), NP)],
                    htp1)

    for a in (acc0, acc1, acc2, acc3):
        @plsc.parallel_loop(0, NP, step=16, unroll=8)
        def _zero_acc(j, a=a):
            a[pl.ds(j, 16)] = jnp.zeros((16,), jnp.float32)

    pltpu.async_copy(epk_hbm.at[pl.ds(0, CW)], ebuf0, sem0)
    pltpu.async_copy(epk_hbm.at[pl.ds(CW, CW)], ebuf1, sem1)

    hi_mask = jnp.full((16,), -65536, jnp.int32)
    lo_mask = jnp.full((16,), 65535, jnp.int32)

    def _pair(g, _):
        for b, (ebuf, sem) in enumerate(((ebuf0, sem0), (ebuf1, sem1))):
            k = 2 * g + b
            pltpu.make_async_copy(epk_hbm.at[pl.ds(0, CW)], ebuf, sem).wait()

            @plsc.parallel_loop(0, CE, step=16, unroll=8)
            def _win(i):
                sd = ebuf[pl.ds(i, 16)]
                wt = plsc.bitcast(ebuf[pl.ds(CE + i, 16)], jnp.float32)
                s = sd & lo_mask
                d = lax.shift_right_logical(sd, 16)
                for htp, accl, acch in ((htp0, acc0, acc1),
                                        (htp1, acc2, acc3)):
                    g2 = plsc.load_gather(htp, [s])
                    lo = plsc.bitcast(lax.shift_left(g2, 16), jnp.float32)
                    hi = plsc.bitcast(g2 & hi_mask, jnp.float32)
                    plsc.addupdate_scatter(accl, [d], lo * wt)
                    plsc.addupdate_scatter(acch, [d], hi * wt)

            off = pl.multiple_of((k + 2) * CW, 8)
            pltpu.async_copy(epk_hbm.at[pl.ds(off, CW)], ebuf, sem)
        return 0
    lax.fori_loop(0, NCH // 2, _pair, 0)
    pltpu.make_async_copy(epk_hbm.at[pl.ds(0, CW)], ebuf0, sem0).wait()
    pltpu.make_async_copy(epk_hbm.at[pl.ds(0, CW)], ebuf1, sem1).wait()

    for a, frow in ((acc0, p0), (acc1, p0 + H // 2),
                    (acc2, p0 + 1), (acc3, p0 + 1 + H // 2)):
        pltpu.sync_copy(a, sums_hbm.at[pl.ds(pl.multiple_of(frow * NP, 8),
                                             NP)])

    pltpu.make_async_copy(dst_hbm.at[pl.ds(0, ECNT)], cdst_buf, semc).wait()

    @plsc.parallel_loop(0, NP, step=16, unroll=8)
    def _zero_cnt(j):
        cnt_buf[pl.ds(j, 16)] = jnp.zeros((16,), jnp.float32)

    ones = jnp.full((16,), 1.0, jnp.float32)

    @plsc.parallel_loop(0, ECNT, step=16, unroll=8)
    def _cwin(i):
        d = cdst_buf[pl.ds(i, 16)]
        plsc.addupdate_scatter(cnt_buf, [d], ones)

    pltpu.sync_copy(cnt_buf, cnt_hbm.at[pl.ds(pl.multiple_of(wid * NP, 8), NP)])


@functools.cache
def _edge_kernel():
    return pl.kernel(
        _edge_body,
        out_type=[jax.ShapeDtypeStruct((D * NP,), jnp.float32),
                  jax.ShapeDtypeStruct((NW * NP,), jnp.float32)],
        mesh=plsc.VectorSubcoreMesh(core_axis_name="c", subcore_axis_name="s",
                                    num_cores=NC, num_subcores=NS),
        compiler_params=pltpu.CompilerParams(needs_layout_passes=False),
        scratch_types=[pltpu.VMEM((NP,), jnp.int32),
                       pltpu.VMEM((NP,), jnp.int32),
                       pltpu.VMEM((NP,), jnp.float32),
                       pltpu.VMEM((NP,), jnp.float32),
                       pltpu.VMEM((NP,), jnp.float32),
                       pltpu.VMEM((NP,), jnp.float32),
                       pltpu.VMEM((CW,), jnp.int32),
                       pltpu.VMEM((CW,), jnp.int32),
                       pltpu.VMEM((NP,), jnp.float32),
                       pltpu.VMEM((ECNT,), jnp.int32),
                       pltpu.SemaphoreType.DMA,
                       pltpu.SemaphoreType.DMA,
                       pltpu.SemaphoreType.DMA])


def _update_body(xt_ref, sums_ref, cntp_ref, uw1xt_ref, uw1at_ref, uw2t_ref,
                 s1x_ref, t1x_ref, s1a_ref, t1a_ref, s2_ref, t2_ref,
                 uc1_ref, uc2_ref, out_ref):
    cnt = jnp.sum(cntp_ref[...], axis=0, keepdims=True)
    agg = sums_ref[...] / jnp.maximum(cnt, 1.0)
    xb = xt_ref[...] * s1x_ref[...] + t1x_ref[...]
    ab = agg * s1a_ref[...] + t1a_ref[...]
    z1 = (jnp.dot(uw1xt_ref[...], xb, preferred_element_type=jnp.float32)
          + jnp.dot(uw1at_ref[...], ab, preferred_element_type=jnp.float32)
          + uc1_ref[...])
    h1 = _gelu(z1)
    hb = h1 * s2_ref[...] + t2_ref[...]
    out_ref[...] = _gelu(jnp.dot(uw2t_ref[...], hb,
                                 preferred_element_type=jnp.float32)
                         + uc2_ref[...])


def _update_ffn_t(xt, sums_t, cntp, uw1xt, uw1at, uw2t,
                  s1x, t1x, s1a, t1a, s2, t2, uc1, uc2):
    col = pl.BlockSpec((H, 1), lambda i: (0, 0))
    full = pl.BlockSpec((H, H), lambda i: (0, 0))
    blk = pl.BlockSpec((D, BLK), lambda i: (0, i))
    return pl.pallas_call(
        _update_body,
        grid=(GRID,),
        in_specs=[blk, blk, pl.BlockSpec((NW, BLK), lambda i: (0, i)),
                  full, full, full,
                  col, col, col, col, col, col, col, col],
        out_specs=pl.BlockSpec((H, BLK), lambda i: (0, i)),
        out_shape=jax.ShapeDtypeStruct((H, NP), jnp.float32),
    )(xt, sums_t, cntp, uw1xt, uw1at, uw2t,
      s1x, t1x, s1a, t1a, s2, t2, uc1, uc2)


def kernel(x, edges, edge_weights, g1, b1, m1, v1, W1, c1, g2, b2, m2, v2,
           W2, c2, ug1, ub1, um1, uv1, UW1, uc1, ug2, ub2, um2, uv2, UW2, uc2):
    xt = jnp.pad(x.T, ((0, 0), (0, NP - N)))

    def colv(p):
        return p.reshape(-1, 1)

    s1, t1 = _bn_scale_shift(g1, b1, m1, v1)
    s2, t2 = _bn_scale_shift(g2, b2, m2, v2)
    ht = _prepare_ffn_t(xt, W1.T, W2.T, colv(s1), colv(t1), colv(s2),
                        colv(t2), colv(c1), colv(c2))

    wbits = lax.bitcast_convert_type(edge_weights, jnp.int32)
    sd = edges[1] | (edges[0] << 16)
    epk = jnp.stack([sd.reshape(NCH, CE), wbits.reshape(NCH, CE)],
                    axis=1).reshape(-1)
    epk = jnp.pad(epk, (0, 2 * CW))
    sums_flat, cnt_flat = _edge_kernel()(ht.reshape(-1), epk, edges[0])

    us1, ut1 = _bn_scale_shift(ug1, ub1, um1, uv1)
    us2, ut2 = _bn_scale_shift(ug2, ub2, um2, uv2)
    out_t = _update_ffn_t(
        xt, sums_flat.reshape(D, NP), cnt_flat.reshape(NW, NP),
        UW1[:D].T, UW1[D:].T, UW2.T,
        colv(us1[:D]), colv(ut1[:D]), colv(us1[D:]), colv(ut1[D:]),
        colv(us2), colv(ut2), colv(uc1), colv(uc2))
    return out_t[:, :N].T


# speedup vs baseline: 1.0597x; 1.0498x over previous
import functools

import jax
import jax.numpy as jnp
from jax import lax
from jax.experimental import pallas as pl
from jax.experimental.pallas import tpu as pltpu, tpu_sc as plsc

N = 10000
D = 128
H = 128
E = 320000
NC = 2
NS = 16
NW = NC * NS
CE = 3200
NCH = E // CE
CW = 2 * CE
ECNT = E // NW
BLK = 1250
GRID = N // BLK


def _fold_bn(g, b, m, v):
    s = g / jnp.sqrt(v + 1e-3)
    return s, b - m * s


def _gelu(z):
    return 0.5 * z * (1.0 + lax.erf(z * 0.7071067811865476))


def _prepare_body(xt_ref, w1t_ref, w2t_ref, bn1_ref, bn2_ref, c1_ref, c2_ref,
                  out_ref):
    s1, t1 = _fold_bn(*(bn1_ref[i] for i in range(4)))
    s2, t2 = _fold_bn(*(bn2_ref[i] for i in range(4)))
    xb = xt_ref[...] * s1[:, None] + t1[:, None]
    h1 = _gelu(jnp.dot(w1t_ref[...], xb, preferred_element_type=jnp.float32)
               + c1_ref[...])
    hb = h1 * s2[:, None] + t2[:, None]
    h = _gelu(jnp.dot(w2t_ref[...], hb, preferred_element_type=jnp.float32)
              + c2_ref[...])
    au = lax.bitcast_convert_type(h[:H // 2].astype(jnp.bfloat16),
                                  jnp.uint16).astype(jnp.int32)
    bu = lax.bitcast_convert_type(h[H // 2:].astype(jnp.bfloat16),
                                  jnp.uint16).astype(jnp.int32)
    out_ref[...] = au | (bu << 16)


def _prepare_ffn_t(xt, w1t, w2t, bn1, bn2, c1, c2):
    return pl.pallas_call(
        _prepare_body,
        out_shape=jax.ShapeDtypeStruct((H // 2, N), jnp.int32),
    )(xt, w1t, w2t, bn1, bn2, c1, c2)


def _edge_body(ht_hbm, epk_hbm, dst_hbm, sums_hbm, cnt_hbm,
               htp0, htp1, acc0, acc1, acc2, acc3, ebuf0, ebuf1,
               cnt_buf, cdst_buf, sem0, sem1, semc):
    wid = lax.axis_index("s") * NC + lax.axis_index("c")
    p0 = 2 * wid

    coff = pl.multiple_of(wid * ECNT, 8)
    pltpu.async_copy(dst_hbm.at[pl.ds(coff, ECNT)], cdst_buf, semc)

    pltpu.sync_copy(ht_hbm.at[pl.ds(pl.multiple_of(p0 * N, 8), N)], htp0)
    pltpu.sync_copy(ht_hbm.at[pl.ds(pl.multiple_of((p0 + 1) * N, 8), N)],
                    htp1)

    for a in (acc0, acc1, acc2, acc3):
        @plsc.parallel_loop(0, N, step=16, unroll=8)
        def _zero_acc(j, a=a):
            a[pl.ds(j, 16)] = jnp.zeros((16,), jnp.float32)

    pltpu.async_copy(epk_hbm.at[pl.ds(0, CW)], ebuf0, sem0)
    pltpu.async_copy(epk_hbm.at[pl.ds(CW, CW)], ebuf1, sem1)

    hi_mask = jnp.full((16,), -65536, jnp.int32)
    lo_mask = jnp.full((16,), 65535, jnp.int32)

    def _pair(g, _):
        for b, (ebuf, sem) in enumerate(((ebuf0, sem0), (ebuf1, sem1))):
            k = 2 * g + b
            pltpu.make_async_copy(epk_hbm.at[pl.ds(0, CW)], ebuf, sem).wait()

            @plsc.parallel_loop(0, CE, step=16, unroll=8)
            def _win(i):
                sd = ebuf[pl.ds(i, 16)]
                wt = plsc.bitcast(ebuf[pl.ds(CE + i, 16)], jnp.float32)
                s = sd & lo_mask
                d = lax.shift_right_logical(sd, 16)
                for htp, accl, acch in ((htp0, acc0, acc1),
                                        (htp1, acc2, acc3)):
                    g2 = plsc.load_gather(htp, [s])
                    lo = plsc.bitcast(lax.shift_left(g2, 16), jnp.float32)
                    hi = plsc.bitcast(g2 & hi_mask, jnp.float32)
                    plsc.addupdate_scatter(accl, [d], lo * wt)
                    plsc.addupdate_scatter(acch, [d], hi * wt)

            off = pl.multiple_of(lax.rem((k + 2) * CW, NCH * CW), 8)
            pltpu.async_copy(epk_hbm.at[pl.ds(off, CW)], ebuf, sem)
        return 0
    lax.fori_loop(0, NCH // 2, _pair, 0)
    pltpu.make_async_copy(epk_hbm.at[pl.ds(0, CW)], ebuf0, sem0).wait()
    pltpu.make_async_copy(epk_hbm.at[pl.ds(0, CW)], ebuf1, sem1).wait()

    for a, frow in ((acc0, p0), (acc1, p0 + H // 2),
                    (acc2, p0 + 1), (acc3, p0 + 1 + H // 2)):
        pltpu.sync_copy(a, sums_hbm.at[pl.ds(pl.multiple_of(frow * N, 8), N)])

    pltpu.make_async_copy(dst_hbm.at[pl.ds(0, ECNT)], cdst_buf, semc).wait()

    @plsc.parallel_loop(0, N, step=16, unroll=8)
    def _zero_cnt(j):
        cnt_buf[pl.ds(j, 16)] = jnp.zeros((16,), jnp.float32)

    ones = jnp.full((16,), 1.0, jnp.float32)

    @plsc.parallel_loop(0, ECNT, step=16, unroll=8)
    def _cwin(i):
        d = cdst_buf[pl.ds(i, 16)]
        plsc.addupdate_scatter(cnt_buf, [d], ones)

    pltpu.sync_copy(cnt_buf, cnt_hbm.at[pl.ds(pl.multiple_of(wid * N, 8), N)])


@functools.cache
def _edge_kernel():
    return pl.kernel(
        _edge_body,
        out_type=[jax.ShapeDtypeStruct((D * N,), jnp.float32),
                  jax.ShapeDtypeStruct((NW * N,), jnp.float32)],
        mesh=plsc.VectorSubcoreMesh(core_axis_name="c", subcore_axis_name="s",
                                    num_cores=NC, num_subcores=NS),
        compiler_params=pltpu.CompilerParams(needs_layout_passes=False),
        scratch_types=[pltpu.VMEM((N,), jnp.int32),
                       pltpu.VMEM((N,), jnp.int32),
                       pltpu.VMEM((N,), jnp.float32),
                       pltpu.VMEM((N,), jnp.float32),
                       pltpu.VMEM((N,), jnp.float32),
                       pltpu.VMEM((N,), jnp.float32),
                       pltpu.VMEM((CW,), jnp.int32),
                       pltpu.VMEM((CW,), jnp.int32),
                       pltpu.VMEM((N,), jnp.float32),
                       pltpu.VMEM((ECNT,), jnp.int32),
                       pltpu.SemaphoreType.DMA,
                       pltpu.SemaphoreType.DMA,
                       pltpu.SemaphoreType.DMA])


def _update_body(xt_ref, sums_ref, cntp_ref, uw1xt_ref, uw1at_ref, uw2t_ref,
                 ubn1_ref, ubn2_ref, uc1_ref, uc2_ref, out_ref):
    s1x, t1x = _fold_bn(*(ubn1_ref[i, :D] for i in range(4)))
    s1a, t1a = _fold_bn(*(ubn1_ref[i, D:] for i in range(4)))
    s2, t2 = _fold_bn(*(ubn2_ref[i] for i in range(4)))
    cnt = jnp.sum(cntp_ref[...], axis=0, keepdims=True)
    agg = sums_ref[...] / jnp.maximum(cnt, 1.0)
    xb = xt_ref[...] * s1x[:, None] + t1x[:, None]
    ab = agg * s1a[:, None] + t1a[:, None]
    z1 = (jnp.dot(uw1xt_ref[...], xb, preferred_element_type=jnp.float32)
          + jnp.dot(uw1at_ref[...], ab, preferred_element_type=jnp.float32)
          + uc1_ref[...])
    h1 = _gelu(z1)
    hb = h1 * s2[:, None] + t2[:, None]
    out_t = _gelu(jnp.dot(uw2t_ref[...], hb,
                          preferred_element_type=jnp.float32) + uc2_ref[...])
    out_ref[...] = out_t.T


def _update_ffn(xt, sums_t, cntp, uw1xt, uw1at, uw2t, ubn1, ubn2, uc1, uc2):
    return pl.pallas_call(
        _update_body,
        out_shape=jax.ShapeDtypeStruct((N, H), jnp.float32),
    )(xt, sums_t, cntp, uw1xt, uw1at, uw2t, ubn1, ubn2, uc1, uc2)


def kernel(x, edges, edge_weights, g1, b1, m1, v1, W1, c1, g2, b2, m2, v2,
           W2, c2, ug1, ub1, um1, uv1, UW1, uc1, ug2, ub2, um2, uv2, UW2, uc2):
    xt = x.T

    def colv(p):
        return p.reshape(-1, 1)

    ht = _prepare_ffn_t(xt, W1.T, W2.T,
                        jnp.stack([g1, b1, m1, v1]),
                        jnp.stack([g2, b2, m2, v2]), colv(c1), colv(c2))

    wbits = lax.bitcast_convert_type(edge_weights, jnp.int32)
    sd = edges[1] | (edges[0] << 16)
    epk = jnp.stack([sd.reshape(NCH, CE), wbits.reshape(NCH, CE)],
                    axis=1).reshape(-1)
    sums_flat, cnt_flat = _edge_kernel()(ht.reshape(-1), epk, edges[0])

    return _update_ffn(
        xt, sums_flat.reshape(D, N), cnt_flat.reshape(NW, N),
        UW1[:D].T, UW1[D:].T, UW2.T,
        jnp.stack([ug1, ub1, um1, uv1]), jnp.stack([ug2, ub2, um2, uv2]),
        colv(uc1), colv(uc2))


# speedup vs baseline: 1.1148x; 1.0520x over previous
import functools

import jax
import jax.numpy as jnp
from jax import lax
from jax.experimental import pallas as pl
from jax.experimental.pallas import tpu as pltpu, tpu_sc as plsc

N = 10000
D = 128
H = 128
E = 320000
NC = 2
NS = 16
NW = NC * NS
CE = 3200
NCH = E // CE
ECNT = E // NW
EROWS = E // 128


def _fold_bn(g, b, m, v):
    s = g / jnp.sqrt(v + 1e-3)
    return s, b - m * s


def _gelu(z):
    return 0.5 * z * (1.0 + lax.erf(z * 0.7071067811865476))


def _prepare_body(x_ref, src_ref, dst_ref, w1_ref, w2_ref,
                  g1_ref, b1_ref, m1_ref, v1_ref, g2_ref, b2_ref, m2_ref,
                  v2_ref, c1_ref, c2_ref, ht_ref, sd_ref):
    s1, t1 = _fold_bn(g1_ref[...], b1_ref[...], m1_ref[...], v1_ref[...])
    s2, t2 = _fold_bn(g2_ref[...], b2_ref[...], m2_ref[...], v2_ref[...])
    xb = x_ref[...] * s1 + t1
    h1 = _gelu(jnp.dot(xb, w1_ref[...], preferred_element_type=jnp.float32)
               + c1_ref[...])
    hb = h1 * s2 + t2
    h = _gelu(jnp.dot(hb, w2_ref[...], preferred_element_type=jnp.float32)
              + c2_ref[...])
    au = lax.bitcast_convert_type(h[:, :H // 2].astype(jnp.bfloat16),
                                  jnp.uint16).astype(jnp.int32)
    bu = lax.bitcast_convert_type(h[:, H // 2:].astype(jnp.bfloat16),
                                  jnp.uint16).astype(jnp.int32)
    ht_ref[...] = (au | (bu << 16)).T
    sd_ref[...] = src_ref[...] | (dst_ref[...] << 16)


def _prepare_ffn(x, src2d, dst2d, w1, w2, bns, c1, c2):
    return pl.pallas_call(
        _prepare_body,
        out_shape=[jax.ShapeDtypeStruct((H // 2, N), jnp.int32),
                   jax.ShapeDtypeStruct((EROWS, 128), jnp.int32)],
    )(x, src2d, dst2d, w1, w2, *bns, c1, c2)


def _edge_body(ht_hbm, sd_hbm, wgt_hbm, dst_hbm, sums_hbm, cnt_hbm,
               htp0, htp1, acc0, acc1, acc2, acc3,
               sdb0, sdb1, wb0, wb1, cnt_buf, cdst_buf, sem0, sem1, semc):
    wid = lax.axis_index("s") * NC + lax.axis_index("c")
    p0 = 2 * wid

    coff = pl.multiple_of(wid * ECNT, 8)
    pltpu.async_copy(dst_hbm.at[pl.ds(coff, ECNT)], cdst_buf, semc)

    pltpu.sync_copy(ht_hbm.at[pl.ds(pl.multiple_of(p0 * N, 8), N)], htp0)
    pltpu.sync_copy(ht_hbm.at[pl.ds(pl.multiple_of((p0 + 1) * N, 8), N)],
                    htp1)

    for a in (acc0, acc1, acc2, acc3):
        @plsc.parallel_loop(0, N, step=16, unroll=8)
        def _zero_acc(j, a=a):
            a[pl.ds(j, 16)] = jnp.zeros((16,), jnp.float32)

    pltpu.async_copy(sd_hbm.at[pl.ds(0, CE)], sdb0, sem0)
    pltpu.async_copy(wgt_hbm.at[pl.ds(0, CE)], wb0, sem0)
    pltpu.async_copy(sd_hbm.at[pl.ds(CE, CE)], sdb1, sem1)
    pltpu.async_copy(wgt_hbm.at[pl.ds(CE, CE)], wb1, sem1)

    hi_mask = jnp.full((16,), -65536, jnp.int32)
    lo_mask = jnp.full((16,), 65535, jnp.int32)

    def _pair(g, _):
        for b, (sdb, wb, sem) in enumerate(((sdb0, wb0, sem0),
                                            (sdb1, wb1, sem1))):
            k = 2 * g + b
            pltpu.make_async_copy(sd_hbm.at[pl.ds(0, CE)], sdb, sem).wait()
            pltpu.make_async_copy(wgt_hbm.at[pl.ds(0, CE)], wb, sem).wait()

            @plsc.parallel_loop(0, CE, step=16, unroll=8)
            def _win(i):
                sd = sdb[pl.ds(i, 16)]
                wt = wb[pl.ds(i, 16)]
                s = sd & lo_mask
                d = lax.shift_right_logical(sd, 16)
                for htp, accl, acch in ((htp0, acc0, acc1),
                                        (htp1, acc2, acc3)):
                    g2 = plsc.load_gather(htp, [s])
                    lo = plsc.bitcast(lax.shift_left(g2, 16), jnp.float32)
                    hi = plsc.bitcast(g2 & hi_mask, jnp.float32)
                    plsc.addupdate_scatter(accl, [d], lo * wt)
                    plsc.addupdate_scatter(acch, [d], hi * wt)

            off = pl.multiple_of(lax.rem((k + 2) * CE, E), 8)
            pltpu.async_copy(sd_hbm.at[pl.ds(off, CE)], sdb, sem)
            pltpu.async_copy(wgt_hbm.at[pl.ds(off, CE)], wb, sem)
        return 0
    lax.fori_loop(0, NCH // 2, _pair, 0)
    pltpu.make_async_copy(sd_hbm.at[pl.ds(0, CE)], sdb0, sem0).wait()
    pltpu.make_async_copy(wgt_hbm.at[pl.ds(0, CE)], wb0, sem0).wait()
    pltpu.make_async_copy(sd_hbm.at[pl.ds(0, CE)], sdb1, sem1).wait()
    pltpu.make_async_copy(wgt_hbm.at[pl.ds(0, CE)], wb1, sem1).wait()

    for a, frow in ((acc0, p0), (acc1, p0 + H // 2),
                    (acc2, p0 + 1), (acc3, p0 + 1 + H // 2)):
        pltpu.sync_copy(a, sums_hbm.at[pl.ds(pl.multiple_of(frow * N, 8), N)])

    pltpu.make_async_copy(dst_hbm.at[pl.ds(0, ECNT)], cdst_buf, semc).wait()

    @plsc.parallel_loop(0, N, step=16, unroll=8)
    def _zero_cnt(j):
        cnt_buf[pl.ds(j, 16)] = jnp.zeros((16,), jnp.float32)

    ones = jnp.full((16,), 1.0, jnp.float32)

    @plsc.parallel_loop(0, ECNT, step=16, unroll=8)
    def _cwin(i):
        d = cdst_buf[pl.ds(i, 16)]
        plsc.addupdate_scatter(cnt_buf, [d], ones)

    pltpu.sync_copy(cnt_buf, cnt_hbm.at[pl.ds(pl.multiple_of(wid * N, 8), N)])


@functools.cache
def _edge_kernel():
    return pl.kernel(
        _edge_body,
        out_type=[jax.ShapeDtypeStruct((D * N,), jnp.float32),
                  jax.ShapeDtypeStruct((NW * N,), jnp.float32)],
        mesh=plsc.VectorSubcoreMesh(core_axis_name="c", subcore_axis_name="s",
                                    num_cores=NC, num_subcores=NS),
        compiler_params=pltpu.CompilerParams(needs_layout_passes=False),
        scratch_types=[pltpu.VMEM((N,), jnp.int32),
                       pltpu.VMEM((N,), jnp.int32),
                       pltpu.VMEM((N,), jnp.float32),
                       pltpu.VMEM((N,), jnp.float32),
                       pltpu.VMEM((N,), jnp.float32),
                       pltpu.VMEM((N,), jnp.float32),
                       pltpu.VMEM((CE,), jnp.int32),
                       pltpu.VMEM((CE,), jnp.int32),
                       pltpu.VMEM((CE,), jnp.float32),
                       pltpu.VMEM((CE,), jnp.float32),
                       pltpu.VMEM((N,), jnp.float32),
                       pltpu.VMEM((ECNT,), jnp.int32),
                       pltpu.SemaphoreType.DMA,
                       pltpu.SemaphoreType.DMA,
                       pltpu.SemaphoreType.DMA])


def _update_body(x_ref, sums_ref, cntp_ref, uw1_ref, uw2_ref,
                 ug1_ref, ub1_ref, um1_ref, uv1_ref, ug2_ref, ub2_ref,
                 um2_ref, uv2_ref, uc1_ref, uc2_ref, out_ref):
    s1, t1 = _fold_bn(ug1_ref[...], ub1_ref[...], um1_ref[...], uv1_ref[...])
    s2, t2 = _fold_bn(ug2_ref[...], ub2_ref[...], um2_ref[...], uv2_ref[...])
    cnt = jnp.sum(cntp_ref[...], axis=0, keepdims=True)
    agg = (sums_ref[...] / jnp.maximum(cnt, 1.0)).T
    xb = x_ref[...] * s1[:, :D] + t1[:, :D]
    ab = agg * s1[:, D:] + t1[:, D:]
    z1 = (jnp.dot(xb, uw1_ref[:D], preferred_element_type=jnp.float32)
          + jnp.dot(ab, uw1_ref[D:], preferred_element_type=jnp.float32)
          + uc1_ref[...])
    h1 = _gelu(z1)
    hb = h1 * s2 + t2
    out_ref[...] = _gelu(jnp.dot(hb, uw2_ref[...],
                                 preferred_element_type=jnp.float32)
                         + uc2_ref[...])


def _update_ffn(x, sums_t, cntp, uw1, uw2, ubns, uc1, uc2):
    return pl.pallas_call(
        _update_body,
        out_shape=jax.ShapeDtypeStruct((N, H), jnp.float32),
    )(x, sums_t, cntp, uw1, uw2, *ubns, uc1, uc2)


def kernel(x, edges, edge_weights, g1, b1, m1, v1, W1, c1, g2, b2, m2, v2,
           W2, c2, ug1, ub1, um1, uv1, UW1, uc1, ug2, ub2, um2, uv2, UW2, uc2):
    def rowv(p):
        return p.reshape(1, -1)

    src2d = edges[1].reshape(EROWS, 128)
    dst2d = edges[0].reshape(EROWS, 128)

    ht, sd = _prepare_ffn(
        x, src2d, dst2d, W1, W2,
        [rowv(p) for p in (g1, b1, m1, v1, g2, b2, m2, v2)],
        rowv(c1), rowv(c2))

    sums_flat, cnt_flat = _edge_kernel()(
        ht.reshape(-1), sd.reshape(-1), edge_weights, edges[0])

    return _update_ffn(
        x, sums_flat.reshape(D, N), cnt_flat.reshape(NW, N), UW1, UW2,
        [rowv(p) for p in (ug1, ub1, um1, uv1, ug2, ub2, um2, uv2)],
        rowv(uc1), rowv(uc2))


# speedup vs baseline: 1.1251x; 1.0093x over previous
import functools

import jax
import jax.numpy as jnp
from jax import lax
from jax.experimental import pallas as pl
from jax.experimental.pallas import tpu as pltpu, tpu_sc as plsc

N = 10000
D = 128
H = 128
E = 320000
NC = 2
NS = 16
NW = NC * NS
CE = 3200
NCH = E // CE
ECNT = E // NW
EROWS = E // 128


def _fold_bn(g, b, m, v):
    s = g / jnp.sqrt(v + 1e-3)
    return s, b - m * s


def _gelu(z):
    return 0.5 * z * (1.0 + lax.erf(z * 0.7071067811865476))


def _prepare_body(x_ref, src_ref, dst_ref, w1_ref, w2_ref,
                  g1_ref, b1_ref, m1_ref, v1_ref, g2_ref, b2_ref, m2_ref,
                  v2_ref, c1_ref, c2_ref, ht_ref, sd_ref):
    s1, t1 = _fold_bn(g1_ref[...], b1_ref[...], m1_ref[...], v1_ref[...])
    s2, t2 = _fold_bn(g2_ref[...], b2_ref[...], m2_ref[...], v2_ref[...])
    xb = x_ref[...] * s1 + t1
    h1 = _gelu(jnp.dot(xb, w1_ref[...], preferred_element_type=jnp.float32)
               + c1_ref[...])
    hb = h1 * s2 + t2
    h = _gelu(jnp.dot(hb, w2_ref[...], preferred_element_type=jnp.float32)
              + c2_ref[...])
    au = lax.bitcast_convert_type(h[:, :H // 2].astype(jnp.bfloat16),
                                  jnp.uint16).astype(jnp.int32)
    bu = lax.bitcast_convert_type(h[:, H // 2:].astype(jnp.bfloat16),
                                  jnp.uint16).astype(jnp.int32)
    ht_ref[...] = (au | (bu << 16)).T
    sd_ref[...] = src_ref[...] | (dst_ref[...] << 16)


def _prepare_ffn(x, src2d, dst2d, w1, w2, bns, c1, c2):
    return pl.pallas_call(
        _prepare_body,
        out_shape=[jax.ShapeDtypeStruct((H // 2, N), jnp.int32),
                   jax.ShapeDtypeStruct((EROWS, 128), jnp.int32)],
    )(x, src2d, dst2d, w1, w2, *bns, c1, c2)


def _edge_body(ht_hbm, sd_hbm, wgt_hbm, dst_hbm, sums_hbm, cnt_hbm,
               htp0, htp1, acc0, acc1, acc2, acc3,
               sdb0, sdb1, wb0, wb1, cnt_buf, cdst_buf, sem0, sem1, semc, semh):
    wid = lax.axis_index("s") * NC + lax.axis_index("c")
    p0 = 2 * wid

    coff = pl.multiple_of(wid * ECNT, 8)
    pltpu.async_copy(dst_hbm.at[pl.ds(coff, ECNT)], cdst_buf, semc)

    pltpu.async_copy(ht_hbm.at[pl.ds(pl.multiple_of(p0 * N, 8), N)], htp0,
                     semh)
    pltpu.async_copy(ht_hbm.at[pl.ds(pl.multiple_of((p0 + 1) * N, 8), N)],
                     htp1, semh)
    pltpu.async_copy(sd_hbm.at[pl.ds(0, CE)], sdb0, sem0)
    pltpu.async_copy(wgt_hbm.at[pl.ds(0, CE)], wb0, sem0)
    pltpu.async_copy(sd_hbm.at[pl.ds(CE, CE)], sdb1, sem1)
    pltpu.async_copy(wgt_hbm.at[pl.ds(CE, CE)], wb1, sem1)

    for a in (acc0, acc1, acc2, acc3):
        @plsc.parallel_loop(0, N, step=16, unroll=8)
        def _zero_acc(j, a=a):
            a[pl.ds(j, 16)] = jnp.zeros((16,), jnp.float32)

    pltpu.make_async_copy(ht_hbm.at[pl.ds(0, N)], htp0, semh).wait()
    pltpu.make_async_copy(ht_hbm.at[pl.ds(0, N)], htp1, semh).wait()

    hi_mask = jnp.full((16,), -65536, jnp.int32)
    lo_mask = jnp.full((16,), 65535, jnp.int32)

    def _pair(g, _):
        for b, (sdb, wb, sem) in enumerate(((sdb0, wb0, sem0),
                                            (sdb1, wb1, sem1))):
            k = 2 * g + b
            pltpu.make_async_copy(sd_hbm.at[pl.ds(0, CE)], sdb, sem).wait()
            pltpu.make_async_copy(wgt_hbm.at[pl.ds(0, CE)], wb, sem).wait()

            @plsc.parallel_loop(0, CE, step=16, unroll=8)
            def _win(i):
                sd = sdb[pl.ds(i, 16)]
                wt = wb[pl.ds(i, 16)]
                s = sd & lo_mask
                d = lax.shift_right_logical(sd, 16)
                for htp, accl, acch in ((htp0, acc0, acc1),
                                        (htp1, acc2, acc3)):
                    g2 = plsc.load_gather(htp, [s])
                    lo = plsc.bitcast(lax.shift_left(g2, 16), jnp.float32)
                    hi = plsc.bitcast(g2 & hi_mask, jnp.float32)
                    plsc.addupdate_scatter(accl, [d], lo * wt)
                    plsc.addupdate_scatter(acch, [d], hi * wt)

            off = pl.multiple_of(lax.rem((k + 2) * CE, E), 8)
            pltpu.async_copy(sd_hbm.at[pl.ds(off, CE)], sdb, sem)
            pltpu.async_copy(wgt_hbm.at[pl.ds(off, CE)], wb, sem)
        return 0
    lax.fori_loop(0, NCH // 2, _pair, 0)
    pltpu.make_async_copy(sd_hbm.at[pl.ds(0, CE)], sdb0, sem0).wait()
    pltpu.make_async_copy(wgt_hbm.at[pl.ds(0, CE)], wb0, sem0).wait()
    pltpu.make_async_copy(sd_hbm.at[pl.ds(0, CE)], sdb1, sem1).wait()
    pltpu.make_async_copy(wgt_hbm.at[pl.ds(0, CE)], wb1, sem1).wait()

    for a, frow in ((acc0, p0), (acc1, p0 + H // 2),
                    (acc2, p0 + 1), (acc3, p0 + 1 + H // 2)):
        pltpu.async_copy(a, sums_hbm.at[pl.ds(pl.multiple_of(frow * N, 8), N)],
                         semh)

    pltpu.make_async_copy(dst_hbm.at[pl.ds(0, ECNT)], cdst_buf, semc).wait()

    @plsc.parallel_loop(0, N, step=16, unroll=8)
    def _zero_cnt(j):
        cnt_buf[pl.ds(j, 16)] = jnp.zeros((16,), jnp.float32)

    ones = jnp.full((16,), 1.0, jnp.float32)

    @plsc.parallel_loop(0, ECNT, step=16, unroll=8)
    def _cwin(i):
        d = cdst_buf[pl.ds(i, 16)]
        plsc.addupdate_scatter(cnt_buf, [d], ones)

    for a in (acc0, acc1, acc2, acc3):
        pltpu.make_async_copy(a, sums_hbm.at[pl.ds(0, N)], semh).wait()
    pltpu.sync_copy(cnt_buf, cnt_hbm.at[pl.ds(pl.multiple_of(wid * N, 8), N)])


@functools.cache
def _edge_kernel():
    return pl.kernel(
        _edge_body,
        out_type=[jax.ShapeDtypeStruct((D * N,), jnp.float32),
                  jax.ShapeDtypeStruct((NW * N,), jnp.float32)],
        mesh=plsc.VectorSubcoreMesh(core_axis_name="c", subcore_axis_name="s",
                                    num_cores=NC, num_subcores=NS),
        compiler_params=pltpu.CompilerParams(needs_layout_passes=False),
        scratch_types=[pltpu.VMEM((N,), jnp.int32),
                       pltpu.VMEM((N,), jnp.int32),
                       pltpu.VMEM((N,), jnp.float32),
                       pltpu.VMEM((N,), jnp.float32),
                       pltpu.VMEM((N,), jnp.float32),
                       pltpu.VMEM((N,), jnp.float32),
                       pltpu.VMEM((CE,), jnp.int32),
                       pltpu.VMEM((CE,), jnp.int32),
                       pltpu.VMEM((CE,), jnp.float32),
                       pltpu.VMEM((CE,), jnp.float32),
                       pltpu.VMEM((N,), jnp.float32),
                       pltpu.VMEM((ECNT,), jnp.int32),
                       pltpu.SemaphoreType.DMA,
                       pltpu.SemaphoreType.DMA,
                       pltpu.SemaphoreType.DMA,
                       pltpu.SemaphoreType.DMA])


def _update_body(x_ref, sums_ref, cntp_ref, uw1_ref, uw2_ref,
                 ug1_ref, ub1_ref, um1_ref, uv1_ref, ug2_ref, ub2_ref,
                 um2_ref, uv2_ref, uc1_ref, uc2_ref, out_ref):
    s1, t1 = _fold_bn(ug1_ref[...], ub1_ref[...], um1_ref[...], uv1_ref[...])
    s2, t2 = _fold_bn(ug2_ref[...], ub2_ref[...], um2_ref[...], uv2_ref[...])
    cnt = jnp.sum(cntp_ref[...], axis=0, keepdims=True)
    agg = (sums_ref[...] / jnp.maximum(cnt, 1.0)).T
    xb = x_ref[...] * s1[:, :D] + t1[:, :D]
    ab = agg * s1[:, D:] + t1[:, D:]
    z1 = (jnp.dot(xb, uw1_ref[:D], preferred_element_type=jnp.float32)
          + jnp.dot(ab, uw1_ref[D:], preferred_element_type=jnp.float32)
          + uc1_ref[...])
    h1 = _gelu(z1)
    hb = h1 * s2 + t2
    out_ref[...] = _gelu(jnp.dot(hb, uw2_ref[...],
                                 preferred_element_type=jnp.float32)
                         + uc2_ref[...])


def _update_ffn(x, sums_t, cntp, uw1, uw2, ubns, uc1, uc2):
    return pl.pallas_call(
        _update_body,
        out_shape=jax.ShapeDtypeStruct((N, H), jnp.float32),
    )(x, sums_t, cntp, uw1, uw2, *ubns, uc1, uc2)


def kernel(x, edges, edge_weights, g1, b1, m1, v1, W1, c1, g2, b2, m2, v2,
           W2, c2, ug1, ub1, um1, uv1, UW1, uc1, ug2, ub2, um2, uv2, UW2, uc2):
    def rowv(p):
        return p.reshape(1, -1)

    src2d = edges[1].reshape(EROWS, 128)
    dst2d = edges[0].reshape(EROWS, 128)

    ht, sd = _prepare_ffn(
        x, src2d, dst2d, W1, W2,
        [rowv(p) for p in (g1, b1, m1, v1, g2, b2, m2, v2)],
        rowv(c1), rowv(c2))

    sums_flat, cnt_flat = _edge_kernel()(
        ht.reshape(-1), sd.reshape(-1), edge_weights, edges[0])

    return _update_ffn(
        x, sums_flat.reshape(D, N), cnt_flat.reshape(NW, N), UW1, UW2,
        [rowv(p) for p in (ug1, ub1, um1, uv1, ug2, ub2, um2, uv2)],
        rowv(uc1), rowv(uc2))
